# Initial kernel scaffold; baseline (speedup 1.0000x reference)
import jax
import jax.numpy as jnp
from jax.experimental import pallas as pl


def kernel(x, edge_index, W1, b1, W2, b2):
    raise NotImplementedError("write your pallas kernel here")


# speedup vs baseline: 15.7210x; 15.7210x over previous
import dataclasses
import functools

import jax
import jax.numpy as jnp
from jax import lax
from jax.experimental import pallas as pl
from jax.experimental.pallas import tpu as pltpu
from jax.experimental.pallas import tpu_sc as plsc

N = 10000
E = 320000
D_IN = 128
D_HID = 256
D_OUT = 128
DH = 128

NC = 2
NS = 16
LANES = 16

NPAD = 10240
W_EDGES = 80
W_CHUNK = 25
NCH1 = E // (NS * W_CHUNK * W_EDGES)
NCH2 = E // (NC * NS * W_CHUNK * W_EDGES)

EPT_DEG = E // (NC * NS)

_mesh = plsc.VectorSubcoreMesh(core_axis_name="c", subcore_axis_name="s")

_sc_params = pltpu.CompilerParams()
if "needs_layout_passes" in pltpu.CompilerParams.__dataclass_fields__:
    _sc_params = dataclasses.replace(_sc_params, needs_layout_passes=False)


_DROWS = NPAD // 128


def _deg_body(dst_hbm, out_hbm, dst_v, degl, ident, deg_sh):
    c = lax.axis_index("c")
    s = lax.axis_index("s")
    t = c * NS + s

    @pl.loop(0, _DROWS)
    def _(r):
        @pl.loop(0, 8)
        def _(k):
            degl[r, pl.ds(k * LANES, LANES)] = jnp.zeros((LANES,), jnp.float32)

    @pl.when(s < 10)
    def _():
        pltpu.sync_copy(degl.at[pl.ds(s * 8, 8)], deg_sh.at[pl.ds(s * 8, 8)])

    @pl.loop(0, 5)
    def _(k):
        ident[0, pl.ds(k * LANES, LANES)] = (
            jnp.arange(LANES, dtype=jnp.int32) + k * LANES
        )

    pltpu.sync_copy(dst_hbm.at[t], dst_v)
    ones = jnp.ones((LANES,), jnp.float32)

    @pl.loop(0, EPT_DEG // LANES)
    def _(i):
        dvec = dst_v[pl.ds(i * LANES, LANES)]
        row = lax.shift_right_logical(dvec, 7)
        col = lax.bitwise_and(dvec, 127)
        plsc.addupdate_scatter(degl, [row, col], ones)

    plsc.subcore_barrier()

    pltpu.sync_copy(degl, deg_sh.at[ident.at[0]], add=True)

    plsc.subcore_barrier()

    @pl.when(s == 0)
    def _():
        pltpu.sync_copy(deg_sh, out_hbm.at[c])


def _sc_degree(dst_r):
    k = pl.kernel(
        _deg_body,
        out_type=jax.ShapeDtypeStruct((NC, _DROWS, 128), jnp.float32),
        mesh=_mesh,
        compiler_params=_sc_params,
        scratch_types=[
            pltpu.VMEM((EPT_DEG,), jnp.int32),
            pltpu.VMEM((_DROWS, 128), jnp.float32),
            pltpu.VMEM((1, 80), jnp.int32),
            pltpu.VMEM_SHARED((_DROWS, 128), jnp.float32),
        ],
    )
    return k(dst_r)


def _agg_body(nch, edge_split, zs_hbm, src_hbm, dst_hbm, out_hbm,
              srcv, dstv, rows, acc):
    c = lax.axis_index("c")
    s = lax.axis_index("s")

    @pl.loop(0, W_EDGES)
    def _(r):
        @pl.loop(0, DH // LANES)
        def _(k):
            rows[r, pl.ds(k * LANES, LANES)] = jnp.zeros((LANES,), jnp.float32)

    @pl.loop(0, 8)
    def _(k):
        pltpu.sync_copy(rows, acc.at[pl.ds(s * 640 + k * W_EDGES, W_EDGES)])

    plsc.subcore_barrier()

    if edge_split:
        src_t = src_hbm.at[c].at[s]
        dst_t = dst_hbm.at[c].at[s]
        z_ref = zs_hbm
    else:
        src_t = src_hbm.at[s]
        dst_t = dst_hbm.at[s]
        z_ref = zs_hbm.at[c]

    @pl.loop(0, nch)
    def _(ch):
        pltpu.sync_copy(src_t.at[ch], srcv)
        pltpu.sync_copy(dst_t.at[ch], dstv)

        @pl.loop(0, W_CHUNK)
        def _(w):
            pltpu.sync_copy(z_ref.at[srcv.at[w]], rows)
            pltpu.sync_copy(rows, acc.at[dstv.at[w]], add=True)

    plsc.subcore_barrier()

    npt = NPAD // NS
    pltpu.sync_copy(
        acc.at[pl.ds(s * npt, npt)], out_hbm.at[c].at[pl.ds(s * npt, npt)]
    )


def _sc_aggregate(zs, src_r, dst_r, nch, edge_split):
    k = pl.kernel(
        functools.partial(_agg_body, nch, edge_split),
        out_type=jax.ShapeDtypeStruct((NC, NPAD, DH), jnp.float32),
        mesh=_mesh,
        compiler_params=_sc_params,
        scratch_types=[
            pltpu.VMEM((W_CHUNK, W_EDGES), jnp.int32),
            pltpu.VMEM((W_CHUNK, W_EDGES), jnp.int32),
            pltpu.VMEM((W_EDGES, DH), jnp.float32),
            pltpu.VMEM_SHARED((NPAD, DH), jnp.float32),
        ],
    )
    return k(zs, src_r, dst_r)


_ROWS = 1000


def _dinv_body(dp_ref, o_ref):
    d = dp_ref[0] + dp_ref[1] + 1.0
    o_ref[...] = lax.rsqrt(d)


def _tc_dinv(deg_parts):
    return pl.pallas_call(
        _dinv_body,
        out_shape=jax.ShapeDtypeStruct((NPAD, 1), jnp.float32),
    )(deg_parts.reshape(NC, NPAD, 1))


def _z1_body(x_ref, w_ref, dinv_ref, o_ref):
    xw = jnp.dot(x_ref[...], w_ref[...], preferred_element_type=jnp.float32)
    z = xw * dinv_ref[...]
    o_ref[0] = z[:, :D_HID // 2]
    o_ref[1] = z[:, D_HID // 2:]


def _tc_z1(x, W1, dinv):
    grid = (N // _ROWS,)
    return pl.pallas_call(
        _z1_body,
        grid=grid,
        in_specs=[
            pl.BlockSpec((_ROWS, D_IN), lambda i: (i, 0)),
            pl.BlockSpec((D_IN, D_HID), lambda i: (0, 0)),
            pl.BlockSpec((_ROWS, 1), lambda i: (i, 0)),
        ],
        out_specs=pl.BlockSpec((NC, _ROWS, D_HID // 2), lambda i: (0, i, 0)),
        out_shape=jax.ShapeDtypeStruct((NC, N, D_HID // 2), jnp.float32),
    )(x, W1, dinv)


def _z2_body(z1_ref, agg1_ref, dinv_ref, w_ref, b_ref, o_ref):
    zf = jnp.concatenate([z1_ref[0], z1_ref[1]], axis=1)
    af = jnp.concatenate([agg1_ref[0], agg1_ref[1]], axis=1)
    h = jax.nn.relu((af + zf) * dinv_ref[...] + b_ref[...])
    z2 = jnp.dot(h, w_ref[...], preferred_element_type=jnp.float32)
    o_ref[...] = z2 * dinv_ref[...]


def _tc_z2(z1, agg1, dinv, W2, b1):
    grid = (N // _ROWS,)
    return pl.pallas_call(
        _z2_body,
        grid=grid,
        in_specs=[
            pl.BlockSpec((NC, _ROWS, D_HID // 2), lambda i: (0, i, 0)),
            pl.BlockSpec((NC, _ROWS, D_HID // 2), lambda i: (0, i, 0)),
            pl.BlockSpec((_ROWS, 1), lambda i: (i, 0)),
            pl.BlockSpec((D_HID, D_OUT), lambda i: (0, 0)),
            pl.BlockSpec((1, D_HID), lambda i: (0, 0)),
        ],
        out_specs=pl.BlockSpec((_ROWS, D_OUT), lambda i: (i, 0)),
        out_shape=jax.ShapeDtypeStruct((N, D_OUT), jnp.float32),
    )(z1, agg1, dinv, W2, b1)


def _final_body(z2_ref, agg2_ref, dinv_ref, b_ref, o_ref):
    af = agg2_ref[0] + agg2_ref[1]
    o_ref[...] = (af + z2_ref[...]) * dinv_ref[...] + b_ref[...]


def _tc_final(z2, agg2, dinv, b2):
    grid = (N // _ROWS,)
    return pl.pallas_call(
        _final_body,
        grid=grid,
        in_specs=[
            pl.BlockSpec((_ROWS, D_OUT), lambda i: (i, 0)),
            pl.BlockSpec((NC, _ROWS, D_OUT), lambda i: (0, i, 0)),
            pl.BlockSpec((_ROWS, 1), lambda i: (i, 0)),
            pl.BlockSpec((1, D_OUT), lambda i: (0, 0)),
        ],
        out_specs=pl.BlockSpec((_ROWS, D_OUT), lambda i: (i, 0)),
        out_shape=jax.ShapeDtypeStruct((N, D_OUT), jnp.float32),
    )(z2, agg2, dinv, b2)


def kernel(x, edge_index, W1, b1, W2, b2):
    src = edge_index[0]
    dst = edge_index[1]
    src_r1 = src.reshape(NS, NCH1, W_CHUNK, W_EDGES)
    dst_r1 = dst.reshape(NS, NCH1, W_CHUNK, W_EDGES)
    src_r2 = src.reshape(NC, NS, NCH2, W_CHUNK, W_EDGES)
    dst_r2 = dst.reshape(NC, NS, NCH2, W_CHUNK, W_EDGES)
    dst_deg = dst.reshape(NC * NS, EPT_DEG)

    deg_parts = _sc_degree(dst_deg)
    dinv = _tc_dinv(deg_parts)

    z1 = _tc_z1(x, W1, dinv)
    agg1 = _sc_aggregate(z1, src_r1, dst_r1, NCH1, edge_split=False)

    z2 = _tc_z2(z1, agg1, dinv, W2, b1.reshape(1, D_HID))
    agg2 = _sc_aggregate(z2, src_r2, dst_r2, NCH2, edge_split=True)

    return _tc_final(z2, agg2, dinv, b2.reshape(1, D_OUT))


# speedup vs baseline: 20.6195x; 1.3116x over previous
import dataclasses
import functools

import jax
import jax.numpy as jnp
from jax import lax
from jax.experimental import pallas as pl
from jax.experimental.pallas import tpu as pltpu
from jax.experimental.pallas import tpu_sc as plsc

N = 10000
E = 320000
D_IN = 128
D_HID = 256
D_OUT = 128
DH = 128

NC = 2
NS = 16
LANES = 16

NPAD = 10240
WE1 = 80
WC1 = 50
NCH1 = E // (NS * WC1 * WE1)
WE2 = 100
WC2 = 20
NCH2 = E // (NC * NS * WC2 * WE2)

EPT_DEG = E // (NC * NS)

_mesh = plsc.VectorSubcoreMesh(core_axis_name="c", subcore_axis_name="s")

_sc_params = pltpu.CompilerParams()
if "needs_layout_passes" in pltpu.CompilerParams.__dataclass_fields__:
    _sc_params = dataclasses.replace(_sc_params, needs_layout_passes=False)


_DROWS = NPAD // 128


def _deg_body(dst_hbm, out_hbm, dst_v, degl, ident, deg_sh):
    c = lax.axis_index("c")
    s = lax.axis_index("s")
    t = c * NS + s

    @pl.loop(0, _DROWS)
    def _(r):
        @pl.loop(0, 8)
        def _(k):
            degl[r, pl.ds(k * LANES, LANES)] = jnp.zeros((LANES,), jnp.float32)

    @pl.when(s < 10)
    def _():
        pltpu.sync_copy(degl.at[pl.ds(s * 8, 8)], deg_sh.at[pl.ds(s * 8, 8)])

    @pl.loop(0, 5)
    def _(k):
        ident[0, pl.ds(k * LANES, LANES)] = (
            jnp.arange(LANES, dtype=jnp.int32) + k * LANES
        )

    pltpu.sync_copy(dst_hbm.at[t], dst_v)
    ones = jnp.ones((LANES,), jnp.float32)

    @pl.loop(0, EPT_DEG // LANES)
    def _(i):
        dvec = dst_v[pl.ds(i * LANES, LANES)]
        row = lax.shift_right_logical(dvec, 7)
        col = lax.bitwise_and(dvec, 127)
        plsc.addupdate_scatter(degl, [row, col], ones)

    plsc.subcore_barrier()

    pltpu.sync_copy(degl, deg_sh.at[ident.at[0]], add=True)

    plsc.subcore_barrier()

    @pl.when(s == 0)
    def _():
        pltpu.sync_copy(deg_sh, out_hbm.at[c])


def _sc_degree(dst_r):
    k = pl.kernel(
        _deg_body,
        out_type=jax.ShapeDtypeStruct((NC, _DROWS, 128), jnp.float32),
        mesh=_mesh,
        compiler_params=_sc_params,
        scratch_types=[
            pltpu.VMEM((EPT_DEG,), jnp.int32),
            pltpu.VMEM((_DROWS, 128), jnp.float32),
            pltpu.VMEM((1, 80), jnp.int32),
            pltpu.VMEM_SHARED((_DROWS, 128), jnp.float32),
        ],
    )
    return k(dst_r)


def _agg_body(nch, wc, we, edge_split, zs_hbm, src_hbm, dst_hbm, out_hbm,
              srcv, dstv, rows_a, rows_b, acc, gs_a, gs_b, ss_a, ss_b):
    c = lax.axis_index("c")
    s = lax.axis_index("s")

    @pl.loop(0, we)
    def _(r):
        @pl.loop(0, DH // LANES)
        def _(k):
            rows_a[r, pl.ds(k * LANES, LANES)] = jnp.zeros((LANES,),
                                                           jnp.float32)

    zrows = NPAD // NS
    @pl.loop(0, zrows // we)
    def _(k):
        pltpu.sync_copy(rows_a, acc.at[pl.ds(s * zrows + k * we, we)])
    if zrows % we:
        pltpu.sync_copy(
            rows_a.at[pl.ds(0, zrows % we)],
            acc.at[pl.ds(s * zrows + (zrows // we) * we, zrows % we)],
        )

    plsc.subcore_barrier()

    if edge_split:
        src_t = src_hbm.at[c].at[s]
        dst_t = dst_hbm.at[c].at[s]
        z_ref = zs_hbm
    else:
        src_t = src_hbm.at[s]
        dst_t = dst_hbm.at[s]
        z_ref = zs_hbm.at[c]

    bufs = (rows_a, rows_b)
    gsems = (gs_a, gs_b)
    ssems = (ss_a, ss_b)

    def start_g(w, b):
        pltpu.async_copy(z_ref.at[srcv.at[w]], bufs[b], gsems[b])

    def wait_g(w, b):
        pltpu.make_async_copy(z_ref.at[srcv.at[w]], bufs[b], gsems[b]).wait()

    def start_s(w, b):
        pltpu.async_copy(bufs[b], acc.at[dstv.at[w]], ssems[b], add=True)

    def wait_s(w, b):
        pltpu.make_async_copy(bufs[b], acc.at[dstv.at[w]], ssems[b]).wait()

    npair = wc // 2

    @pl.loop(0, nch)
    def _(ch):
        pltpu.sync_copy(src_t.at[ch], srcv)
        pltpu.sync_copy(dst_t.at[ch], dstv)
        start_g(0, 0)
        start_g(1, 1)

        @pl.loop(0, npair - 1)
        def _(j):
            w = j * 2
            wait_g(w, 0)
            start_s(w, 0)
            wait_g(w + 1, 1)
            start_s(w + 1, 1)
            wait_s(w, 0)
            start_g(w + 2, 0)
            wait_s(w + 1, 1)
            start_g(w + 3, 1)

        w = wc - 2
        wait_g(w, 0)
        start_s(w, 0)
        wait_g(w + 1, 1)
        start_s(w + 1, 1)
        wait_s(w, 0)
        wait_s(w + 1, 1)

    plsc.subcore_barrier()

    npt = NPAD // NS
    pltpu.sync_copy(
        acc.at[pl.ds(s * npt, npt)], out_hbm.at[c].at[pl.ds(s * npt, npt)]
    )


def _sc_aggregate(zs, src_r, dst_r, nch, wc, we, edge_split):
    k = pl.kernel(
        functools.partial(_agg_body, nch, wc, we, edge_split),
        out_type=jax.ShapeDtypeStruct((NC, NPAD, DH), jnp.float32),
        mesh=_mesh,
        compiler_params=_sc_params,
        scratch_types=[
            pltpu.VMEM((wc, we), jnp.int32),
            pltpu.VMEM((wc, we), jnp.int32),
            pltpu.VMEM((we, DH), jnp.float32),
            pltpu.VMEM((we, DH), jnp.float32),
            pltpu.VMEM_SHARED((NPAD, DH), jnp.float32),
            pltpu.SemaphoreType.DMA,
            pltpu.SemaphoreType.DMA,
            pltpu.SemaphoreType.DMA,
            pltpu.SemaphoreType.DMA,
        ],
    )
    return k(zs, src_r, dst_r)


_ROWS = 1000


def _dinv_body(dp_ref, o_ref):
    d = dp_ref[0] + dp_ref[1] + 1.0
    o_ref[...] = lax.rsqrt(d)


def _tc_dinv(deg_parts):
    return pl.pallas_call(
        _dinv_body,
        out_shape=jax.ShapeDtypeStruct((NPAD, 1), jnp.float32),
    )(deg_parts.reshape(NC, NPAD, 1))


def _z1_body(x_ref, w_ref, dinv_ref, o_ref):
    xw = jnp.dot(x_ref[...], w_ref[...], preferred_element_type=jnp.float32)
    z = xw * dinv_ref[...]
    o_ref[0] = z[:, :D_HID // 2]
    o_ref[1] = z[:, D_HID // 2:]


def _tc_z1(x, W1, dinv):
    grid = (N // _ROWS,)
    return pl.pallas_call(
        _z1_body,
        grid=grid,
        in_specs=[
            pl.BlockSpec((_ROWS, D_IN), lambda i: (i, 0)),
            pl.BlockSpec((D_IN, D_HID), lambda i: (0, 0)),
            pl.BlockSpec((_ROWS, 1), lambda i: (i, 0)),
        ],
        out_specs=pl.BlockSpec((NC, _ROWS, D_HID // 2), lambda i: (0, i, 0)),
        out_shape=jax.ShapeDtypeStruct((NC, N, D_HID // 2), jnp.float32),
    )(x, W1, dinv)


def _z2_body(z1_ref, agg1_ref, dinv_ref, w_ref, b_ref, o_ref):
    zf = jnp.concatenate([z1_ref[0], z1_ref[1]], axis=1)
    af = jnp.concatenate([agg1_ref[0], agg1_ref[1]], axis=1)
    h = jax.nn.relu((af + zf) * dinv_ref[...] + b_ref[...])
    z2 = jnp.dot(h, w_ref[...], preferred_element_type=jnp.float32)
    o_ref[...] = z2 * dinv_ref[...]


def _tc_z2(z1, agg1, dinv, W2, b1):
    grid = (N // _ROWS,)
    return pl.pallas_call(
        _z2_body,
        grid=grid,
        in_specs=[
            pl.BlockSpec((NC, _ROWS, D_HID // 2), lambda i: (0, i, 0)),
            pl.BlockSpec((NC, _ROWS, D_HID // 2), lambda i: (0, i, 0)),
            pl.BlockSpec((_ROWS, 1), lambda i: (i, 0)),
            pl.BlockSpec((D_HID, D_OUT), lambda i: (0, 0)),
            pl.BlockSpec((1, D_HID), lambda i: (0, 0)),
        ],
        out_specs=pl.BlockSpec((_ROWS, D_OUT), lambda i: (i, 0)),
        out_shape=jax.ShapeDtypeStruct((N, D_OUT), jnp.float32),
    )(z1, agg1, dinv, W2, b1)


def _final_body(z2_ref, agg2_ref, dinv_ref, b_ref, o_ref):
    af = agg2_ref[0] + agg2_ref[1]
    o_ref[...] = (af + z2_ref[...]) * dinv_ref[...] + b_ref[...]


def _tc_final(z2, agg2, dinv, b2):
    grid = (N // _ROWS,)
    return pl.pallas_call(
        _final_body,
        grid=grid,
        in_specs=[
            pl.BlockSpec((_ROWS, D_OUT), lambda i: (i, 0)),
            pl.BlockSpec((NC, _ROWS, D_OUT), lambda i: (0, i, 0)),
            pl.BlockSpec((_ROWS, 1), lambda i: (i, 0)),
            pl.BlockSpec((1, D_OUT), lambda i: (0, 0)),
        ],
        out_specs=pl.BlockSpec((_ROWS, D_OUT), lambda i: (i, 0)),
        out_shape=jax.ShapeDtypeStruct((N, D_OUT), jnp.float32),
    )(z2, agg2, dinv, b2)


def kernel(x, edge_index, W1, b1, W2, b2):
    src = edge_index[0]
    dst = edge_index[1]
    src_r1 = src.reshape(NS, NCH1, WC1, WE1)
    dst_r1 = dst.reshape(NS, NCH1, WC1, WE1)
    src_r2 = src.reshape(NC, NS, NCH2, WC2, WE2)
    dst_r2 = dst.reshape(NC, NS, NCH2, WC2, WE2)
    dst_deg = dst.reshape(NC * NS, EPT_DEG)

    deg_parts = _sc_degree(dst_deg)
    dinv = _tc_dinv(deg_parts)

    z1 = _tc_z1(x, W1, dinv)
    agg1 = _sc_aggregate(z1, src_r1, dst_r1, NCH1, WC1, WE1, edge_split=False)

    z2 = _tc_z2(z1, agg1, dinv, W2, b1.reshape(1, D_HID))
    agg2 = _sc_aggregate(z2, src_r2, dst_r2, NCH2, WC2, WE2, edge_split=True)

    return _tc_final(z2, agg2, dinv, b2.reshape(1, D_OUT))


# speedup vs baseline: 24.6072x; 1.1934x over previous
import dataclasses
import functools

import jax
import jax.numpy as jnp
from jax import lax
from jax.experimental import pallas as pl
from jax.experimental.pallas import tpu as pltpu
from jax.experimental.pallas import tpu_sc as plsc

N = 10000
E = 320000
D_IN = 128
D_HID = 256
D_OUT = 128
DH = 128

NC = 2
NS = 16
LANES = 16

NPAD = 10240
WE1 = 50
WC1 = 40
NB1 = 4
NCH1 = E // (NS * WC1 * WE1)
WE2 = 50
WC2 = 40
NB2 = 4
NCH2 = E // (NC * NS * WC2 * WE2)

EPT_DEG = E // (NC * NS)

_mesh = plsc.VectorSubcoreMesh(core_axis_name="c", subcore_axis_name="s")

_sc_params = pltpu.CompilerParams()
if "needs_layout_passes" in pltpu.CompilerParams.__dataclass_fields__:
    _sc_params = dataclasses.replace(_sc_params, needs_layout_passes=False)


_DROWS = NPAD // 128


def _deg_body(dst_hbm, out_hbm, dst_v, degl, ident, deg_sh):
    c = lax.axis_index("c")
    s = lax.axis_index("s")
    t = c * NS + s

    @pl.loop(0, _DROWS)
    def _(r):
        @pl.loop(0, 8)
        def _(k):
            degl[r, pl.ds(k * LANES, LANES)] = jnp.zeros((LANES,), jnp.float32)

    @pl.when(s < 10)
    def _():
        pltpu.sync_copy(degl.at[pl.ds(s * 8, 8)], deg_sh.at[pl.ds(s * 8, 8)])

    @pl.loop(0, 5)
    def _(k):
        ident[0, pl.ds(k * LANES, LANES)] = (
            jnp.arange(LANES, dtype=jnp.int32) + k * LANES
        )

    pltpu.sync_copy(dst_hbm.at[t], dst_v)
    ones = jnp.ones((LANES,), jnp.float32)

    @pl.loop(0, EPT_DEG // LANES)
    def _(i):
        dvec = dst_v[pl.ds(i * LANES, LANES)]
        row = lax.shift_right_logical(dvec, 7)
        col = lax.bitwise_and(dvec, 127)
        plsc.addupdate_scatter(degl, [row, col], ones)

    plsc.subcore_barrier()

    pltpu.sync_copy(degl, deg_sh.at[ident.at[0]], add=True)

    plsc.subcore_barrier()

    @pl.when(s == 0)
    def _():
        pltpu.sync_copy(deg_sh, out_hbm.at[c])


def _sc_degree(dst_r):
    k = pl.kernel(
        _deg_body,
        out_type=jax.ShapeDtypeStruct((NC, _DROWS, 128), jnp.float32),
        mesh=_mesh,
        compiler_params=_sc_params,
        scratch_types=[
            pltpu.VMEM((EPT_DEG,), jnp.int32),
            pltpu.VMEM((_DROWS, 128), jnp.float32),
            pltpu.VMEM((1, 80), jnp.int32),
            pltpu.VMEM_SHARED((_DROWS, 128), jnp.float32),
        ],
    )
    return k(dst_r)


def _agg_body(nch, wc, we, nbuf, edge_split, zs_hbm, src_hbm, dst_hbm,
              out_hbm, srcv, dstv, *rest):
    bufs = rest[:nbuf]
    gsems = rest[nbuf:2 * nbuf]
    ssems = rest[2 * nbuf:3 * nbuf]
    acc = rest[3 * nbuf]

    c = lax.axis_index("c")
    s = lax.axis_index("s")

    @pl.loop(0, we)
    def _(r):
        @pl.loop(0, DH // LANES)
        def _(k):
            bufs[0][r, pl.ds(k * LANES, LANES)] = jnp.zeros((LANES,),
                                                            jnp.float32)

    zrows = NPAD // NS
    @pl.loop(0, zrows // 32)
    def _(k):
        pltpu.sync_copy(
            bufs[0].at[pl.ds(0, 32)], acc.at[pl.ds(s * zrows + k * 32, 32)]
        )

    plsc.subcore_barrier()

    if edge_split:
        src_t = src_hbm.at[c].at[s]
        dst_t = dst_hbm.at[c].at[s]
        z_ref = zs_hbm
    else:
        src_t = src_hbm.at[s]
        dst_t = dst_hbm.at[s]
        z_ref = zs_hbm.at[c]

    def start_g(w, b):
        pltpu.async_copy(z_ref.at[srcv.at[w]], bufs[b], gsems[b])

    def wait_g(w, b):
        pltpu.make_async_copy(z_ref.at[srcv.at[w]], bufs[b], gsems[b]).wait()

    def start_s(w, b):
        pltpu.async_copy(bufs[b], acc.at[dstv.at[w]], ssems[b], add=True)

    def wait_s(w, b):
        pltpu.make_async_copy(bufs[b], acc.at[dstv.at[w]], ssems[b]).wait()

    ngroups = wc // nbuf

    @pl.loop(0, nch)
    def _(ch):
        pltpu.sync_copy(src_t.at[ch], srcv)
        pltpu.sync_copy(dst_t.at[ch], dstv)
        for b in range(nbuf):
            start_g(b, b)

        @pl.loop(0, ngroups - 1)
        def _(j):
            w0 = j * nbuf
            for b in range(nbuf):
                wait_g(w0 + b, b)
                start_s(w0 + b, b)
            for b in range(nbuf):
                wait_s(w0 + b, b)
                start_g(w0 + nbuf + b, b)

        w0 = wc - nbuf
        for b in range(nbuf):
            wait_g(w0 + b, b)
            start_s(w0 + b, b)
        for b in range(nbuf):
            wait_s(w0 + b, b)

    plsc.subcore_barrier()

    npt = NPAD // NS
    pltpu.sync_copy(
        acc.at[pl.ds(s * npt, npt)], out_hbm.at[c].at[pl.ds(s * npt, npt)]
    )


def _sc_aggregate(zs, src_r, dst_r, nch, wc, we, nbuf, edge_split):
    k = pl.kernel(
        functools.partial(_agg_body, nch, wc, we, nbuf, edge_split),
        out_type=jax.ShapeDtypeStruct((NC, NPAD, DH), jnp.float32),
        mesh=_mesh,
        compiler_params=_sc_params,
        scratch_types=(
            [
                pltpu.VMEM((wc, we), jnp.int32),
                pltpu.VMEM((wc, we), jnp.int32),
            ]
            + [pltpu.VMEM((we, DH), jnp.float32)] * nbuf
            + [pltpu.SemaphoreType.DMA] * (2 * nbuf)
            + [pltpu.VMEM_SHARED((NPAD, DH), jnp.float32)]
        ),
    )
    return k(zs, src_r, dst_r)


_ROWS = 1000


def _dinv_body(dp_ref, o_ref):
    d = dp_ref[0] + dp_ref[1] + 1.0
    o_ref[...] = lax.rsqrt(d)


def _tc_dinv(deg_parts):
    return pl.pallas_call(
        _dinv_body,
        out_shape=jax.ShapeDtypeStruct((NPAD, 1), jnp.float32),
    )(deg_parts.reshape(NC, NPAD, 1))


def _z1_body(x_ref, w_ref, dinv_ref, o_ref):
    xw = jnp.dot(x_ref[...], w_ref[...], preferred_element_type=jnp.float32)
    z = xw * dinv_ref[...]
    o_ref[0] = z[:, :D_HID // 2]
    o_ref[1] = z[:, D_HID // 2:]


def _tc_z1(x, W1, dinv):
    grid = (N // _ROWS,)
    return pl.pallas_call(
        _z1_body,
        grid=grid,
        in_specs=[
            pl.BlockSpec((_ROWS, D_IN), lambda i: (i, 0)),
            pl.BlockSpec((D_IN, D_HID), lambda i: (0, 0)),
            pl.BlockSpec((_ROWS, 1), lambda i: (i, 0)),
        ],
        out_specs=pl.BlockSpec((NC, _ROWS, D_HID // 2), lambda i: (0, i, 0)),
        out_shape=jax.ShapeDtypeStruct((NC, N, D_HID // 2), jnp.float32),
    )(x, W1, dinv)


def _z2_body(z1_ref, agg1_ref, dinv_ref, w_ref, b_ref, o_ref):
    zf = jnp.concatenate([z1_ref[0], z1_ref[1]], axis=1)
    af = jnp.concatenate([agg1_ref[0], agg1_ref[1]], axis=1)
    h = jax.nn.relu((af + zf) * dinv_ref[...] + b_ref[...])
    z2 = jnp.dot(h, w_ref[...], preferred_element_type=jnp.float32)
    o_ref[...] = z2 * dinv_ref[...]


def _tc_z2(z1, agg1, dinv, W2, b1):
    grid = (N // _ROWS,)
    return pl.pallas_call(
        _z2_body,
        grid=grid,
        in_specs=[
            pl.BlockSpec((NC, _ROWS, D_HID // 2), lambda i: (0, i, 0)),
            pl.BlockSpec((NC, _ROWS, D_HID // 2), lambda i: (0, i, 0)),
            pl.BlockSpec((_ROWS, 1), lambda i: (i, 0)),
            pl.BlockSpec((D_HID, D_OUT), lambda i: (0, 0)),
            pl.BlockSpec((1, D_HID), lambda i: (0, 0)),
        ],
        out_specs=pl.BlockSpec((_ROWS, D_OUT), lambda i: (i, 0)),
        out_shape=jax.ShapeDtypeStruct((N, D_OUT), jnp.float32),
    )(z1, agg1, dinv, W2, b1)


def _final_body(z2_ref, agg2_ref, dinv_ref, b_ref, o_ref):
    af = agg2_ref[0] + agg2_ref[1]
    o_ref[...] = (af + z2_ref[...]) * dinv_ref[...] + b_ref[...]


def _tc_final(z2, agg2, dinv, b2):
    grid = (N // _ROWS,)
    return pl.pallas_call(
        _final_body,
        grid=grid,
        in_specs=[
            pl.BlockSpec((_ROWS, D_OUT), lambda i: (i, 0)),
            pl.BlockSpec((NC, _ROWS, D_OUT), lambda i: (0, i, 0)),
            pl.BlockSpec((_ROWS, 1), lambda i: (i, 0)),
            pl.BlockSpec((1, D_OUT), lambda i: (0, 0)),
        ],
        out_specs=pl.BlockSpec((_ROWS, D_OUT), lambda i: (i, 0)),
        out_shape=jax.ShapeDtypeStruct((N, D_OUT), jnp.float32),
    )(z2, agg2, dinv, b2)


def kernel(x, edge_index, W1, b1, W2, b2):
    src = edge_index[0]
    dst = edge_index[1]
    src_r1 = src.reshape(NS, NCH1, WC1, WE1)
    dst_r1 = dst.reshape(NS, NCH1, WC1, WE1)
    src_r2 = src.reshape(NC, NS, NCH2, WC2, WE2)
    dst_r2 = dst.reshape(NC, NS, NCH2, WC2, WE2)
    dst_deg = dst.reshape(NC * NS, EPT_DEG)

    deg_parts = _sc_degree(dst_deg)
    dinv = _tc_dinv(deg_parts)

    z1 = _tc_z1(x, W1, dinv)
    agg1 = _sc_aggregate(z1, src_r1, dst_r1, NCH1, WC1, WE1, NB1,
                         edge_split=False)

    z2 = _tc_z2(z1, agg1, dinv, W2, b1.reshape(1, D_HID))
    agg2 = _sc_aggregate(z2, src_r2, dst_r2, NCH2, WC2, WE2, NB2,
                         edge_split=True)

    return _tc_final(z2, agg2, dinv, b2.reshape(1, D_OUT))


# speedup vs baseline: 25.2347x; 1.0255x over previous
import dataclasses
import functools

import jax
import jax.numpy as jnp
from jax import lax
from jax.experimental import pallas as pl
from jax.experimental.pallas import tpu as pltpu
from jax.experimental.pallas import tpu_sc as plsc

N = 10000
E = 320000
D_IN = 128
D_HID = 256
D_OUT = 128
DH = 128

NC = 2
NS = 16
LANES = 16

NPAD = 10240
WE1 = 50
WC1 = 40
NB1 = 5
NCH1 = E // (NS * WC1 * WE1)
WE2 = 50
WC2 = 40
NB2 = 5
NCH2 = E // (NC * NS * WC2 * WE2)

EPT_DEG = E // (NC * NS)

_mesh = plsc.VectorSubcoreMesh(core_axis_name="c", subcore_axis_name="s")

_sc_params = pltpu.CompilerParams()
if "needs_layout_passes" in pltpu.CompilerParams.__dataclass_fields__:
    _sc_params = dataclasses.replace(_sc_params, needs_layout_passes=False)


_DROWS = NPAD // 128


def _deg_body(dst_hbm, out_hbm, dst_v, degl, ident, deg_sh):
    c = lax.axis_index("c")
    s = lax.axis_index("s")
    t = c * NS + s

    @pl.loop(0, _DROWS)
    def _(r):
        @pl.loop(0, 8)
        def _(k):
            degl[r, pl.ds(k * LANES, LANES)] = jnp.zeros((LANES,), jnp.float32)

    @pl.when(s < 10)
    def _():
        pltpu.sync_copy(degl.at[pl.ds(s * 8, 8)], deg_sh.at[pl.ds(s * 8, 8)])

    @pl.loop(0, 5)
    def _(k):
        ident[0, pl.ds(k * LANES, LANES)] = (
            jnp.arange(LANES, dtype=jnp.int32) + k * LANES
        )

    pltpu.sync_copy(dst_hbm.at[t], dst_v)
    ones = jnp.ones((LANES,), jnp.float32)

    @pl.loop(0, EPT_DEG // LANES)
    def _(i):
        dvec = dst_v[pl.ds(i * LANES, LANES)]
        row = lax.shift_right_logical(dvec, 7)
        col = lax.bitwise_and(dvec, 127)
        plsc.addupdate_scatter(degl, [row, col], ones)

    plsc.subcore_barrier()

    pltpu.sync_copy(degl, deg_sh.at[ident.at[0]], add=True)

    plsc.subcore_barrier()

    @pl.when(s == 0)
    def _():
        pltpu.sync_copy(deg_sh, out_hbm.at[c])


def _sc_degree(dst_r):
    k = pl.kernel(
        _deg_body,
        out_type=jax.ShapeDtypeStruct((NC, _DROWS, 128), jnp.float32),
        mesh=_mesh,
        compiler_params=_sc_params,
        scratch_types=[
            pltpu.VMEM((EPT_DEG,), jnp.int32),
            pltpu.VMEM((_DROWS, 128), jnp.float32),
            pltpu.VMEM((1, 80), jnp.int32),
            pltpu.VMEM_SHARED((_DROWS, 128), jnp.float32),
        ],
    )
    return k(dst_r)


def _agg_body(nch, wc, we, nbuf, edge_split, zs_hbm, src_hbm, dst_hbm,
              out_hbm, srcv, dstv, *rest):
    bufs = rest[:nbuf]
    gsems = rest[nbuf:2 * nbuf]
    ssems = rest[2 * nbuf:3 * nbuf]
    acc = rest[3 * nbuf]

    c = lax.axis_index("c")
    s = lax.axis_index("s")

    @pl.loop(0, we)
    def _(r):
        @pl.loop(0, DH // LANES)
        def _(k):
            bufs[0][r, pl.ds(k * LANES, LANES)] = jnp.zeros((LANES,),
                                                            jnp.float32)

    zrows = NPAD // NS
    @pl.loop(0, zrows // 32)
    def _(k):
        pltpu.sync_copy(
            bufs[0].at[pl.ds(0, 32)], acc.at[pl.ds(s * zrows + k * 32, 32)]
        )

    plsc.subcore_barrier()

    if edge_split:
        src_t = src_hbm.at[c].at[s]
        dst_t = dst_hbm.at[c].at[s]
        z_ref = zs_hbm
    else:
        src_t = src_hbm.at[s]
        dst_t = dst_hbm.at[s]
        z_ref = zs_hbm.at[c]

    def start_g(w, b):
        pltpu.async_copy(z_ref.at[srcv.at[w]], bufs[b], gsems[b])

    def wait_g(w, b):
        pltpu.make_async_copy(z_ref.at[srcv.at[w]], bufs[b], gsems[b]).wait()

    def start_s(w, b):
        pltpu.async_copy(bufs[b], acc.at[dstv.at[w]], ssems[b], add=True)

    def wait_s(w, b):
        pltpu.make_async_copy(bufs[b], acc.at[dstv.at[w]], ssems[b]).wait()

    ngroups = wc // nbuf

    @pl.loop(0, nch)
    def _(ch):
        pltpu.sync_copy(src_t.at[ch], srcv)
        pltpu.sync_copy(dst_t.at[ch], dstv)
        for b in range(nbuf):
            start_g(b, b)

        @pl.loop(0, ngroups - 1)
        def _(j):
            w0 = j * nbuf
            for b in range(nbuf):
                wait_g(w0 + b, b)
                start_s(w0 + b, b)
            for b in range(nbuf):
                wait_s(w0 + b, b)
                start_g(w0 + nbuf + b, b)

        w0 = wc - nbuf
        for b in range(nbuf):
            wait_g(w0 + b, b)
            start_s(w0 + b, b)
        for b in range(nbuf):
            wait_s(w0 + b, b)

    plsc.subcore_barrier()

    npt = NPAD // NS
    pltpu.sync_copy(
        acc.at[pl.ds(s * npt, npt)], out_hbm.at[c].at[pl.ds(s * npt, npt)]
    )


def _sc_aggregate(zs, src_r, dst_r, nch, wc, we, nbuf, edge_split):
    k = pl.kernel(
        functools.partial(_agg_body, nch, wc, we, nbuf, edge_split),
        out_type=jax.ShapeDtypeStruct((NC, NPAD, DH), jnp.float32),
        mesh=_mesh,
        compiler_params=_sc_params,
        scratch_types=(
            [
                pltpu.VMEM((wc, we), jnp.int32),
                pltpu.VMEM((wc, we), jnp.int32),
            ]
            + [pltpu.VMEM((we, DH), jnp.float32)] * nbuf
            + [pltpu.SemaphoreType.DMA] * (2 * nbuf)
            + [pltpu.VMEM_SHARED((NPAD, DH), jnp.float32)]
        ),
    )
    return k(zs, src_r, dst_r)


_ROWS = 1000


def _dinv_block(dp_ref):
    return lax.rsqrt(dp_ref[0] + dp_ref[1] + 1.0)


_DEG_SPEC = pl.BlockSpec((NC, _ROWS, 1), lambda i: (0, i, 0))


def _z1_body(x_ref, w_ref, dp_ref, o_ref):
    xw = jnp.dot(x_ref[...], w_ref[...], preferred_element_type=jnp.float32)
    z = xw * _dinv_block(dp_ref)
    o_ref[0] = z[:, :D_HID // 2]
    o_ref[1] = z[:, D_HID // 2:]


def _tc_z1(x, W1, deg_r):
    grid = (N // _ROWS,)
    return pl.pallas_call(
        _z1_body,
        grid=grid,
        in_specs=[
            pl.BlockSpec((_ROWS, D_IN), lambda i: (i, 0)),
            pl.BlockSpec((D_IN, D_HID), lambda i: (0, 0)),
            _DEG_SPEC,
        ],
        out_specs=pl.BlockSpec((NC, _ROWS, D_HID // 2), lambda i: (0, i, 0)),
        out_shape=jax.ShapeDtypeStruct((NC, N, D_HID // 2), jnp.float32),
    )(x, W1, deg_r)


def _z2_body(z1_ref, agg1_ref, dp_ref, w_ref, b_ref, o_ref):
    dinv = _dinv_block(dp_ref)
    zf = jnp.concatenate([z1_ref[0], z1_ref[1]], axis=1)
    af = jnp.concatenate([agg1_ref[0], agg1_ref[1]], axis=1)
    h = jax.nn.relu((af + zf) * dinv + b_ref[...])
    z2 = jnp.dot(h, w_ref[...], preferred_element_type=jnp.float32)
    o_ref[...] = z2 * dinv


def _tc_z2(z1, agg1, deg_r, W2, b1):
    grid = (N // _ROWS,)
    return pl.pallas_call(
        _z2_body,
        grid=grid,
        in_specs=[
            pl.BlockSpec((NC, _ROWS, D_HID // 2), lambda i: (0, i, 0)),
            pl.BlockSpec((NC, _ROWS, D_HID // 2), lambda i: (0, i, 0)),
            _DEG_SPEC,
            pl.BlockSpec((D_HID, D_OUT), lambda i: (0, 0)),
            pl.BlockSpec((1, D_HID), lambda i: (0, 0)),
        ],
        out_specs=pl.BlockSpec((_ROWS, D_OUT), lambda i: (i, 0)),
        out_shape=jax.ShapeDtypeStruct((N, D_OUT), jnp.float32),
    )(z1, agg1, deg_r, W2, b1)


def _final_body(z2_ref, agg2_ref, dp_ref, b_ref, o_ref):
    af = agg2_ref[0] + agg2_ref[1]
    o_ref[...] = (af + z2_ref[...]) * _dinv_block(dp_ref) + b_ref[...]


def _tc_final(z2, agg2, deg_r, b2):
    grid = (N // _ROWS,)
    return pl.pallas_call(
        _final_body,
        grid=grid,
        in_specs=[
            pl.BlockSpec((_ROWS, D_OUT), lambda i: (i, 0)),
            pl.BlockSpec((NC, _ROWS, D_OUT), lambda i: (0, i, 0)),
            _DEG_SPEC,
            pl.BlockSpec((1, D_OUT), lambda i: (0, 0)),
        ],
        out_specs=pl.BlockSpec((_ROWS, D_OUT), lambda i: (i, 0)),
        out_shape=jax.ShapeDtypeStruct((N, D_OUT), jnp.float32),
    )(z2, agg2, deg_r, b2)


def kernel(x, edge_index, W1, b1, W2, b2):
    src = edge_index[0]
    dst = edge_index[1]
    src_r1 = src.reshape(NS, NCH1, WC1, WE1)
    dst_r1 = dst.reshape(NS, NCH1, WC1, WE1)
    src_r2 = src.reshape(NC, NS, NCH2, WC2, WE2)
    dst_r2 = dst.reshape(NC, NS, NCH2, WC2, WE2)
    dst_deg = dst.reshape(NC * NS, EPT_DEG)

    deg_parts = _sc_degree(dst_deg)
    deg_r = deg_parts.reshape(NC, NPAD, 1)

    z1 = _tc_z1(x, W1, deg_r)
    agg1 = _sc_aggregate(z1, src_r1, dst_r1, NCH1, WC1, WE1, NB1,
                         edge_split=False)

    z2 = _tc_z2(z1, agg1, deg_r, W2, b1.reshape(1, D_HID))
    agg2 = _sc_aggregate(z2, src_r2, dst_r2, NCH2, WC2, WE2, NB2,
                         edge_split=True)

    return _tc_final(z2, agg2, deg_r, b2.reshape(1, D_OUT))


# speedup vs baseline: 25.7445x; 1.0202x over previous
import dataclasses
import functools

import jax
import jax.numpy as jnp
from jax import lax
from jax.experimental import pallas as pl
from jax.experimental.pallas import tpu as pltpu
from jax.experimental.pallas import tpu_sc as plsc

N = 10000
E = 320000
D_IN = 128
D_HID = 256
D_OUT = 128
DH = 128

NC = 2
NS = 16
LANES = 16

NPAD = 10240
WE1 = 50
WC1 = 40
NB1 = 4
NCH1 = E // (NS * WC1 * WE1)
WE2 = 50
WC2 = 20
NB2 = 4
NCH2 = E // (NC * NS * WC2 * WE2)

EPT_DEG = E // (NC * NS)

_mesh = plsc.VectorSubcoreMesh(core_axis_name="c", subcore_axis_name="s")

_sc_params = pltpu.CompilerParams()
if "needs_layout_passes" in pltpu.CompilerParams.__dataclass_fields__:
    _sc_params = dataclasses.replace(_sc_params, needs_layout_passes=False)


_DROWS = NPAD // 128


def _deg_body(dst_hbm, out_hbm, dst_v, degl, ident, deg_sh):
    c = lax.axis_index("c")
    s = lax.axis_index("s")
    t = c * NS + s

    @pl.loop(0, _DROWS)
    def _(r):
        @pl.loop(0, 8)
        def _(k):
            degl[r, pl.ds(k * LANES, LANES)] = jnp.zeros((LANES,), jnp.float32)

    @pl.when(s < 10)
    def _():
        pltpu.sync_copy(degl.at[pl.ds(s * 8, 8)], deg_sh.at[pl.ds(s * 8, 8)])

    @pl.loop(0, 5)
    def _(k):
        ident[0, pl.ds(k * LANES, LANES)] = (
            jnp.arange(LANES, dtype=jnp.int32) + k * LANES
        )

    pltpu.sync_copy(dst_hbm.at[t], dst_v)
    ones = jnp.ones((LANES,), jnp.float32)

    @pl.loop(0, EPT_DEG // LANES)
    def _(i):
        dvec = dst_v[pl.ds(i * LANES, LANES)]
        row = lax.shift_right_logical(dvec, 7)
        col = lax.bitwise_and(dvec, 127)
        plsc.addupdate_scatter(degl, [row, col], ones)

    plsc.subcore_barrier()

    pltpu.sync_copy(degl, deg_sh.at[ident.at[0]], add=True)

    plsc.subcore_barrier()

    @pl.when(s == 0)
    def _():
        pltpu.sync_copy(deg_sh, out_hbm.at[c])


def _sc_degree(dst_r):
    k = pl.kernel(
        _deg_body,
        out_type=jax.ShapeDtypeStruct((NC, _DROWS, 128), jnp.float32),
        mesh=_mesh,
        compiler_params=_sc_params,
        scratch_types=[
            pltpu.VMEM((EPT_DEG,), jnp.int32),
            pltpu.VMEM((_DROWS, 128), jnp.float32),
            pltpu.VMEM((1, 80), jnp.int32),
            pltpu.VMEM_SHARED((_DROWS, 128), jnp.float32),
        ],
    )
    return k(dst_r)


def _agg_body(nch, wc, we, nbuf, edge_split, zs_hbm, src_hbm, dst_hbm,
              out_hbm, srcv0, dstv0, srcv1, dstv1, is0, is1, *rest):
    bufs = rest[:nbuf]
    gsems = rest[nbuf:2 * nbuf]
    ssems = rest[2 * nbuf:3 * nbuf]
    acc = rest[3 * nbuf]

    c = lax.axis_index("c")
    s = lax.axis_index("s")

    @pl.loop(0, we)
    def _(r):
        @pl.loop(0, DH // LANES)
        def _(k):
            bufs[0][r, pl.ds(k * LANES, LANES)] = jnp.zeros((LANES,),
                                                            jnp.float32)

    zrows = NPAD // NS
    @pl.loop(0, zrows // 32)
    def _(k):
        pltpu.sync_copy(
            bufs[0].at[pl.ds(0, 32)], acc.at[pl.ds(s * zrows + k * 32, 32)]
        )

    plsc.subcore_barrier()

    if edge_split:
        src_t = src_hbm.at[c].at[s]
        dst_t = dst_hbm.at[c].at[s]
        z_ref = zs_hbm
    else:
        src_t = src_hbm.at[s]
        dst_t = dst_hbm.at[s]
        z_ref = zs_hbm.at[c]

    def start_g(srcv, w, b):
        pltpu.async_copy(z_ref.at[srcv.at[w]], bufs[b], gsems[b])

    def wait_g(srcv, w, b):
        pltpu.make_async_copy(z_ref.at[srcv.at[w]], bufs[b], gsems[b]).wait()

    def start_s(dstv, w, b):
        pltpu.async_copy(bufs[b], acc.at[dstv.at[w]], ssems[b], add=True)

    def wait_s(dstv, w, b):
        pltpu.make_async_copy(bufs[b], acc.at[dstv.at[w]], ssems[b]).wait()

    def start_stage(ch, srcv, dstv, isem):
        pltpu.async_copy(src_t.at[ch], srcv, isem)
        pltpu.async_copy(dst_t.at[ch], dstv, isem)

    def wait_stage(ch, srcv, dstv, isem):
        pltpu.make_async_copy(src_t.at[ch], srcv, isem).wait()
        pltpu.make_async_copy(dst_t.at[ch], dstv, isem).wait()

    ngroups = wc // nbuf

    def do_chunk(srcv, dstv):
        for b in range(nbuf):
            start_g(srcv, b, b)

        @pl.loop(0, ngroups - 1)
        def _(j):
            w0 = j * nbuf
            for b in range(nbuf):
                wait_g(srcv, w0 + b, b)
                start_s(dstv, w0 + b, b)
            for b in range(nbuf):
                wait_s(dstv, w0 + b, b)
                start_g(srcv, w0 + nbuf + b, b)

        w0 = wc - nbuf
        for b in range(nbuf):
            wait_g(srcv, w0 + b, b)
            start_s(dstv, w0 + b, b)
        for b in range(nbuf):
            wait_s(dstv, w0 + b, b)

    nhalf = nch // 2
    start_stage(0, srcv0, dstv0, is0)
    start_stage(1, srcv1, dstv1, is1)

    @pl.loop(0, nhalf)
    def _(m):
        ch = m * 2
        wait_stage(ch, srcv0, dstv0, is0)
        do_chunk(srcv0, dstv0)

        @pl.when(m + 1 < nhalf)
        def _():
            start_stage(ch + 2, srcv0, dstv0, is0)

        wait_stage(ch + 1, srcv1, dstv1, is1)
        do_chunk(srcv1, dstv1)

        @pl.when(m + 1 < nhalf)
        def _():
            start_stage(ch + 3, srcv1, dstv1, is1)

    plsc.subcore_barrier()

    npt = NPAD // NS
    pltpu.sync_copy(
        acc.at[pl.ds(s * npt, npt)], out_hbm.at[c].at[pl.ds(s * npt, npt)]
    )


def _sc_aggregate(zs, src_r, dst_r, nch, wc, we, nbuf, edge_split):
    k = pl.kernel(
        functools.partial(_agg_body, nch, wc, we, nbuf, edge_split),
        out_type=jax.ShapeDtypeStruct((NC, NPAD, DH), jnp.float32),
        mesh=_mesh,
        compiler_params=_sc_params,
        scratch_types=(
            [pltpu.VMEM((wc, we), jnp.int32)] * 4
            + [pltpu.SemaphoreType.DMA] * 2
            + [pltpu.VMEM((we, DH), jnp.float32)] * nbuf
            + [pltpu.SemaphoreType.DMA] * (2 * nbuf)
            + [pltpu.VMEM_SHARED((NPAD, DH), jnp.float32)]
        ),
    )
    return k(zs, src_r, dst_r)


_ROWS = 1000


def _dinv_block(dp_ref):
    return lax.rsqrt(dp_ref[0] + dp_ref[1] + 1.0)


_DEG_SPEC = pl.BlockSpec((NC, _ROWS, 1), lambda i: (0, i, 0))


def _z1_body(x_ref, w_ref, dp_ref, o_ref):
    xw = jnp.dot(x_ref[...], w_ref[...], preferred_element_type=jnp.float32)
    z = xw * _dinv_block(dp_ref)
    o_ref[0] = z[:, :D_HID // 2]
    o_ref[1] = z[:, D_HID // 2:]


def _tc_z1(x, W1, deg_r):
    grid = (N // _ROWS,)
    return pl.pallas_call(
        _z1_body,
        grid=grid,
        in_specs=[
            pl.BlockSpec((_ROWS, D_IN), lambda i: (i, 0)),
            pl.BlockSpec((D_IN, D_HID), lambda i: (0, 0)),
            _DEG_SPEC,
        ],
        out_specs=pl.BlockSpec((NC, _ROWS, D_HID // 2), lambda i: (0, i, 0)),
        out_shape=jax.ShapeDtypeStruct((NC, N, D_HID // 2), jnp.float32),
    )(x, W1, deg_r)


def _z2_body(z1_ref, agg1_ref, dp_ref, w_ref, b_ref, o_ref):
    dinv = _dinv_block(dp_ref)
    zf = jnp.concatenate([z1_ref[0], z1_ref[1]], axis=1)
    af = jnp.concatenate([agg1_ref[0], agg1_ref[1]], axis=1)
    h = jax.nn.relu((af + zf) * dinv + b_ref[...])
    z2 = jnp.dot(h, w_ref[...], preferred_element_type=jnp.float32)
    o_ref[...] = z2 * dinv


def _tc_z2(z1, agg1, deg_r, W2, b1):
    grid = (N // _ROWS,)
    return pl.pallas_call(
        _z2_body,
        grid=grid,
        in_specs=[
            pl.BlockSpec((NC, _ROWS, D_HID // 2), lambda i: (0, i, 0)),
            pl.BlockSpec((NC, _ROWS, D_HID // 2), lambda i: (0, i, 0)),
            _DEG_SPEC,
            pl.BlockSpec((D_HID, D_OUT), lambda i: (0, 0)),
            pl.BlockSpec((1, D_HID), lambda i: (0, 0)),
        ],
        out_specs=pl.BlockSpec((_ROWS, D_OUT), lambda i: (i, 0)),
        out_shape=jax.ShapeDtypeStruct((N, D_OUT), jnp.float32),
    )(z1, agg1, deg_r, W2, b1)


def _final_body(z2_ref, agg2_ref, dp_ref, b_ref, o_ref):
    af = agg2_ref[0] + agg2_ref[1]
    o_ref[...] = (af + z2_ref[...]) * _dinv_block(dp_ref) + b_ref[...]


def _tc_final(z2, agg2, deg_r, b2):
    grid = (N // _ROWS,)
    return pl.pallas_call(
        _final_body,
        grid=grid,
        in_specs=[
            pl.BlockSpec((_ROWS, D_OUT), lambda i: (i, 0)),
            pl.BlockSpec((NC, _ROWS, D_OUT), lambda i: (0, i, 0)),
            _DEG_SPEC,
            pl.BlockSpec((1, D_OUT), lambda i: (0, 0)),
        ],
        out_specs=pl.BlockSpec((_ROWS, D_OUT), lambda i: (i, 0)),
        out_shape=jax.ShapeDtypeStruct((N, D_OUT), jnp.float32),
    )(z2, agg2, deg_r, b2)


def kernel(x, edge_index, W1, b1, W2, b2):
    src = edge_index[0]
    dst = edge_index[1]
    src_r1 = src.reshape(NS, NCH1, WC1, WE1)
    dst_r1 = dst.reshape(NS, NCH1, WC1, WE1)
    src_r2 = src.reshape(NC, NS, NCH2, WC2, WE2)
    dst_r2 = dst.reshape(NC, NS, NCH2, WC2, WE2)
    dst_deg = dst.reshape(NC * NS, EPT_DEG)

    deg_parts = _sc_degree(dst_deg)
    deg_r = deg_parts.reshape(NC, NPAD, 1)

    z1 = _tc_z1(x, W1, deg_r)
    agg1 = _sc_aggregate(z1, src_r1, dst_r1, NCH1, WC1, WE1, NB1,
                         edge_split=False)

    z2 = _tc_z2(z1, agg1, deg_r, W2, b1.reshape(1, D_HID))
    agg2 = _sc_aggregate(z2, src_r2, dst_r2, NCH2, WC2, WE2, NB2,
                         edge_split=True)

    return _tc_final(z2, agg2, deg_r, b2.reshape(1, D_OUT))
